# Initial kernel scaffold; baseline (speedup 1.0000x reference)
import jax
import jax.numpy as jnp
from jax.experimental import pallas as pl


def kernel(x, table, W, b):
    raise NotImplementedError("write your pallas kernel here")


# speedup vs baseline: 2.9398x; 2.9398x over previous
import functools

import jax
import jax.numpy as jnp
from jax import lax
from jax.experimental import pallas as pl
from jax.experimental.pallas import tpu as pltpu
from jax.experimental.pallas import tpu_sc as plsc

_NUM_WORKERS = 32
_CHUNK = 256
_K0 = 128


def _sc_bag_sums(x, table):
  B, L = x.shape
  _, H = table.shape
  bpw = B // _NUM_WORKERS
  n_chunks = bpw // _CHUNK
  k1 = L - _K0
  nc = H // 16
  mesh = plsc.VectorSubcoreMesh(core_axis_name="c", subcore_axis_name="s")

  @functools.partial(
      pl.kernel,
      mesh=mesh,
      compiler_params=pltpu.CompilerParams(use_tc_tiling_on_sc=False),
      out_type=jax.ShapeDtypeStruct((B, H), jnp.float32),
      scratch_types=[
          pltpu.VMEM((_CHUNK, L), jnp.int32),
          pltpu.VMEM((2, L, H), jnp.float32),
          pltpu.VMEM((_CHUNK, H), jnp.float32),
          pltpu.SemaphoreType.DMA,
          pltpu.SemaphoreType.DMA,
      ],
  )
  def sc_kernel(x_hbm, tab_hbm, out_hbm, idx_v, rows_v, acc_v, sem0, sem1):
    wid = lax.axis_index("s") * 2 + lax.axis_index("c")
    base = wid * bpw
    sems = (sem0, sem1)

    def fire(g, slot):
      pltpu.async_copy(tab_hbm.at[idx_v.at[g, pl.ds(0, _K0)]],
                       rows_v.at[slot, pl.ds(0, _K0)], sems[slot])
      pltpu.async_copy(tab_hbm.at[idx_v.at[g, pl.ds(_K0, k1)]],
                       rows_v.at[slot, pl.ds(_K0, k1)], sems[slot])

    def drain(g, slot):
      pltpu.make_async_copy(tab_hbm.at[idx_v.at[g, pl.ds(0, _K0)]],
                            rows_v.at[slot, pl.ds(0, _K0)], sems[slot]).wait()
      pltpu.make_async_copy(tab_hbm.at[idx_v.at[g, pl.ds(_K0, k1)]],
                            rows_v.at[slot, pl.ds(_K0, k1)], sems[slot]).wait()

    for ch in range(n_chunks):
      cbase = base + ch * _CHUNK
      pltpu.sync_copy(x_hbm.at[pl.ds(cbase, _CHUNK)], idx_v)
      fire(0, 0)
      fire(1, 1)

      @pl.loop(0, _CHUNK, step=2)
      def _pair(g2):
        for slot in range(2):
          g = g2 + slot
          drain(g, slot)

          def body(r, accs):
            return tuple(accs[c] + rows_v[slot, r, pl.ds(c * 16, 16)]
                         for c in range(nc))

          zero = jnp.zeros((16,), jnp.float32)
          accs = lax.fori_loop(0, L, body, (zero,) * nc, unroll=4)
          for c in range(nc):
            acc_v[g, pl.ds(c * 16, 16)] = accs[c]

          @pl.when(g + 2 < _CHUNK)
          def _():
            fire(g + 2, slot)

      pltpu.sync_copy(acc_v, out_hbm.at[pl.ds(cbase, _CHUNK)])

  return sc_kernel(x, table)


def _tc_linear(sums, W, b, L):
  B, H = sums.shape
  BM = 2048

  def mm_body(s_ref, w_ref, b_ref, o_ref):
    o_ref[...] = lax.dot_general(
        s_ref[...] * (1.0 / L), w_ref[...],
        (((1,), (1,)), ((), ())),
        preferred_element_type=jnp.float32) + b_ref[...]

  return pl.pallas_call(
      mm_body,
      grid=(B // BM,),
      in_specs=[
          pl.BlockSpec((BM, H), lambda i: (i, 0)),
          pl.BlockSpec((H, H), lambda i: (0, 0)),
          pl.BlockSpec((1, H), lambda i: (0, 0)),
      ],
      out_specs=pl.BlockSpec((BM, H), lambda i: (i, 0)),
      out_shape=jax.ShapeDtypeStruct((B, H), jnp.float32),
  )(sums, W, b[None, :])


def kernel(x, table, W, b):
  L = x.shape[1]
  sums = _sc_bag_sums(x.astype(jnp.int32), table)
  return _tc_linear(sums, W, b, L)


# speedup vs baseline: 3.2939x; 1.1204x over previous
import functools

import jax
import jax.numpy as jnp
from jax import lax
from jax.experimental import pallas as pl
from jax.experimental.pallas import tpu as pltpu
from jax.experimental.pallas import tpu_sc as plsc

_NUM_WORKERS = 32
_CHUNK = 256
_K0 = 128
_BN = 1024


def _tc_project_pack(tT, tail, ws2):
  H, V = tT.shape
  nblk = -(-V // (2 * _BN))
  vtail = V - (nblk - 1) * 2 * _BN

  def body(t_ref, tail_ref, w_ref, o_ref):
    i = pl.program_id(0)

    def dot_t(blk):
      return lax.dot_general(blk, w_ref[...], (((0,), (0,)), ((), ())),
                             preferred_element_type=jnp.float32)

    o_ref[:, 0:H] = dot_t(t_ref[:, 0:_BN])
    o_ref[:, H:2 * H] = dot_t(t_ref[:, _BN:2 * _BN])

    @pl.when(i == nblk - 1)
    def _():
      o_ref[0:vtail, 0:H] = dot_t(tail_ref[...])[_BN - vtail:_BN, :]

  return pl.pallas_call(
      body,
      grid=(nblk,),
      in_specs=[
          pl.BlockSpec((H, 2 * _BN), lambda i: (0, i)),
          pl.BlockSpec((H, _BN), lambda i: (0, 0)),
          pl.BlockSpec((H, H), lambda i: (0, 0)),
      ],
      out_specs=pl.BlockSpec((_BN, 2 * H), lambda i: (i, 0)),
      out_shape=jax.ShapeDtypeStruct((nblk * _BN, 2 * H), jnp.float32),
  )(tT, tail, ws2)


def _sc_gather_pool(x, t_lin, bias):
  B, L = x.shape
  _, H = t_lin.shape
  bpw = B // _NUM_WORKERS
  n_chunks = bpw // _CHUNK
  k1 = L - _K0
  nc = H // 16
  nv = -(-L // 16)
  mesh = plsc.VectorSubcoreMesh(core_axis_name="c", subcore_axis_name="s")

  @functools.partial(
      pl.kernel,
      mesh=mesh,
      compiler_params=pltpu.CompilerParams(use_tc_tiling_on_sc=False),
      out_type=jax.ShapeDtypeStruct((B, H), jnp.float32),
      scratch_types=[
          pltpu.VMEM((_CHUNK, L), jnp.int32),
          pltpu.VMEM((2, L), jnp.int32),
          pltpu.VMEM((2, L, H), jnp.float32),
          pltpu.VMEM((_CHUNK, H), jnp.float32),
          pltpu.VMEM((H,), jnp.float32),
          pltpu.SemaphoreType.DMA,
          pltpu.SemaphoreType.DMA,
      ],
  )
  def sc_kernel(x_hbm, tab_hbm, b_hbm, out_hbm,
                idx_v, idx2_v, rows_v, acc_v, b_v, sem0, sem1):
    wid = lax.axis_index("s") * 2 + lax.axis_index("c")
    base = wid * bpw
    sems = (sem0, sem1)
    pltpu.sync_copy(b_hbm, b_v)
    bias_r = [b_v[pl.ds(c * 16, 16)] for c in range(nc)]

    def fire(g, slot):
      for v in range(nv):
        s = min(v * 16, L - 16)
        r = idx_v[g, pl.ds(s, 16)]
        r2 = ((r & ~jnp.int32(2047))
              | ((r & jnp.int32(1023)) << 1)
              | ((r >> 10) & jnp.int32(1)))
        idx2_v[slot, pl.ds(s, 16)] = r2
      pltpu.async_copy(tab_hbm.at[idx2_v.at[slot, pl.ds(0, _K0)]],
                       rows_v.at[slot, pl.ds(0, _K0)], sems[slot])
      pltpu.async_copy(tab_hbm.at[idx2_v.at[slot, pl.ds(_K0, k1)]],
                       rows_v.at[slot, pl.ds(_K0, k1)], sems[slot])

    def drain(slot):
      pltpu.make_async_copy(tab_hbm.at[idx2_v.at[slot, pl.ds(0, _K0)]],
                            rows_v.at[slot, pl.ds(0, _K0)], sems[slot]).wait()
      pltpu.make_async_copy(tab_hbm.at[idx2_v.at[slot, pl.ds(_K0, k1)]],
                            rows_v.at[slot, pl.ds(_K0, k1)], sems[slot]).wait()

    for ch in range(n_chunks):
      cbase = base + ch * _CHUNK
      pltpu.sync_copy(x_hbm.at[pl.ds(cbase, _CHUNK)], idx_v)
      fire(0, 0)
      fire(1, 1)

      @pl.loop(0, _CHUNK, step=2)
      def _pair(g2):
        for slot in range(2):
          g = g2 + slot
          drain(slot)

          def body(r, accs):
            return tuple(accs[c] + rows_v[slot, r, pl.ds(c * 16, 16)]
                         for c in range(nc))

          accs = lax.fori_loop(0, L, body, tuple(bias_r), unroll=4)
          for c in range(nc):
            acc_v[g, pl.ds(c * 16, 16)] = accs[c]

          @pl.when(g + 2 < _CHUNK)
          def _():
            fire(g + 2, slot)

      pltpu.sync_copy(acc_v, out_hbm.at[pl.ds(cbase, _CHUNK)])

  return sc_kernel(x, t_lin, bias)


def kernel(x, table, W, b):
  B, L = x.shape
  V, H = table.shape
  tT = table.T
  tail = lax.slice(tT, (0, V - _BN), (H, V))
  ws2 = W.T * (1.0 / L)
  packed = _tc_project_pack(tT, tail, ws2)
  t_lin = packed.reshape(-1, H)
  return _sc_gather_pool(x.astype(jnp.int32), t_lin, b)


# speedup vs baseline: 4.1368x; 1.2559x over previous
import functools

import jax
import jax.numpy as jnp
from jax import lax
from jax.experimental import pallas as pl
from jax.experimental.pallas import tpu as pltpu
from jax.experimental.pallas import tpu_sc as plsc

_NUM_WORKERS = 32
_CHUNK = 256
_K0 = 128
_BN = 4096


def _tc_project_pack(tT, tail, ws2):
  H, V = tT.shape
  nblk = -(-V // (2 * _BN))
  vtail = V - (nblk - 1) * 2 * _BN

  def body(t_ref, tail_ref, w_ref, o_ref):
    i = pl.program_id(0)

    def dot_t(blk):
      return lax.dot_general(blk, w_ref[...], (((0,), (0,)), ((), ())),
                             preferred_element_type=jnp.float32)

    o_ref[:, 0:H] = dot_t(t_ref[:, 0:_BN])
    o_ref[:, H:2 * H] = dot_t(t_ref[:, _BN:2 * _BN])

    @pl.when(i == nblk - 1)
    def _():
      o_ref[0:vtail, 0:H] = dot_t(tail_ref[...])[_BN - vtail:_BN, :]

  return pl.pallas_call(
      body,
      grid=(nblk,),
      in_specs=[
          pl.BlockSpec((H, 2 * _BN), lambda i: (0, i)),
          pl.BlockSpec((H, _BN), lambda i: (0, 0)),
          pl.BlockSpec((H, H), lambda i: (0, 0)),
      ],
      out_specs=pl.BlockSpec((_BN, 2 * H), lambda i: (i, 0)),
      out_shape=jax.ShapeDtypeStruct((nblk * _BN, 2 * H), jnp.float32),
  )(tT, tail, ws2)


def _sc_gather_pool(x, t_lin, bias):
  B, L = x.shape
  _, H = t_lin.shape
  bpw = B // _NUM_WORKERS
  n_chunks = bpw // _CHUNK
  k1 = L - _K0
  nc = H // 16
  nv = -(-L // 16)
  mesh = plsc.VectorSubcoreMesh(core_axis_name="c", subcore_axis_name="s")

  @functools.partial(
      pl.kernel,
      mesh=mesh,
      compiler_params=pltpu.CompilerParams(use_tc_tiling_on_sc=False),
      out_type=jax.ShapeDtypeStruct((B, H), jnp.float32),
      scratch_types=[
          pltpu.VMEM((_CHUNK, L), jnp.int32),
          pltpu.VMEM((2, L), jnp.int32),
          pltpu.VMEM((2, L, H), jnp.float32),
          pltpu.VMEM((_CHUNK, H), jnp.float32),
          pltpu.VMEM((H,), jnp.float32),
          pltpu.SemaphoreType.DMA,
          pltpu.SemaphoreType.DMA,
      ],
  )
  def sc_kernel(x_hbm, tab_hbm, b_hbm, out_hbm,
                idx_v, idx2_v, rows_v, acc_v, b_v, sem0, sem1):
    wid = lax.axis_index("s") * 2 + lax.axis_index("c")
    base = wid * bpw
    sems = (sem0, sem1)
    pltpu.sync_copy(b_hbm, b_v)
    bias_r = [b_v[pl.ds(c * 16, 16)] for c in range(nc)]

    def fire(g, slot):
      for v in range(nv):
        s = min(v * 16, L - 16)
        r = idx_v[g, pl.ds(s, 16)]
        r2 = ((r & ~jnp.int32(2 * _BN - 1))
              | ((r & jnp.int32(_BN - 1)) << 1)
              | ((r >> _BN.bit_length() - 1) & jnp.int32(1)))
        idx2_v[slot, pl.ds(s, 16)] = r2
      pltpu.async_copy(tab_hbm.at[idx2_v.at[slot, pl.ds(0, _K0)]],
                       rows_v.at[slot, pl.ds(0, _K0)], sems[slot])
      pltpu.async_copy(tab_hbm.at[idx2_v.at[slot, pl.ds(_K0, k1)]],
                       rows_v.at[slot, pl.ds(_K0, k1)], sems[slot])

    def drain(slot):
      pltpu.make_async_copy(tab_hbm.at[idx2_v.at[slot, pl.ds(0, _K0)]],
                            rows_v.at[slot, pl.ds(0, _K0)], sems[slot]).wait()
      pltpu.make_async_copy(tab_hbm.at[idx2_v.at[slot, pl.ds(_K0, k1)]],
                            rows_v.at[slot, pl.ds(_K0, k1)], sems[slot]).wait()

    for ch in range(n_chunks):
      cbase = base + ch * _CHUNK
      pltpu.sync_copy(x_hbm.at[pl.ds(cbase, _CHUNK)], idx_v)
      fire(0, 0)
      fire(1, 1)

      @pl.loop(0, _CHUNK, step=2)
      def _pair(g2):
        for slot in range(2):
          g = g2 + slot
          drain(slot)

          def body(r, accs):
            return tuple(accs[c] + rows_v[slot, r, pl.ds(c * 16, 16)]
                         for c in range(nc))

          accs = lax.fori_loop(0, L, body, tuple(bias_r), unroll=4)
          for c in range(nc):
            acc_v[g, pl.ds(c * 16, 16)] = accs[c]

          @pl.when(g + 2 < _CHUNK)
          def _():
            fire(g + 2, slot)

      pltpu.sync_copy(acc_v, out_hbm.at[pl.ds(cbase, _CHUNK)])

  return sc_kernel(x, t_lin, bias)


def kernel(x, table, W, b):
  B, L = x.shape
  V, H = table.shape
  tT = table.T
  tail = lax.slice(tT, (0, V - _BN), (H, V))
  ws2 = W.T * (1.0 / L)
  packed = _tc_project_pack(tT, tail, ws2)
  t_lin = packed.reshape(-1, H)
  return _sc_gather_pool(x.astype(jnp.int32), t_lin, b)
